# Initial kernel scaffold; baseline (speedup 1.0000x reference)
import jax
import jax.numpy as jnp
from jax.experimental import pallas as pl


def kernel(prec_weights, orig_prec_fea, self_fea_idx, nbr_fea_idx, reaction_prec_idx, actions_padded, actions_len, prec_elem_mask, params):
    raise NotImplementedError("write your pallas kernel here")


# speedup vs baseline: 1.6516x; 1.6516x over previous
import functools

import jax
import jax.numpy as jnp
from jax import lax
from jax.experimental import pallas as pl
from jax.experimental.pallas import tpu as pltpu
from jax.experimental.pallas import tpu_sc as plsc

F32 = jnp.float32
I32 = jnp.int32

_N = 10000
_E = 320000
_C = 2000
_FEA = 128
_AF = 32
_H = 3
_HID = 256
_GW = _H * _HID
_TW = 2 * _GW
_NBW = _TW + 128
_PW = _GW + 128
_ECH = _E // 16
_NCH = _N // 16
_NWORK = 32


def _mm_body(x_ref, w_ref, b_ref, o_ref):
    o_ref[...] = jnp.dot(x_ref[...], w_ref[...],
                         preferred_element_type=F32) + b_ref[...]


def _mm(x, w, b, bm):
    m, k = x.shape
    n = w.shape[1]
    grid = m // bm
    return pl.pallas_call(
        _mm_body,
        grid=(grid,),
        in_specs=[pl.BlockSpec((bm, k), lambda i: (i, 0)),
                  pl.BlockSpec((k, n), lambda i: (0, 0)),
                  pl.BlockSpec((1, n), lambda i: (0, 0))],
        out_specs=pl.BlockSpec((bm, n), lambda i: (i, 0)),
        out_shape=jax.ShapeDtypeStruct((m, n), F32),
    )(x, w, b.reshape(1, n))


def _gru_body(x_ref, len_ref, wih_ref, whh_ref, bih_ref, bhh_ref, o_ref):
    cdim = x_ref.shape[0]
    h = jnp.zeros((cdim, _AF), F32)
    out = jnp.zeros((cdim, _AF), F32)
    idx = jnp.clip(len_ref[...] - 1, 0, 9)
    wih = wih_ref[...]
    whh = whh_ref[...]
    bih = bih_ref[...]
    bhh = bhh_ref[...]
    for t in range(10):
        xt = x_ref[:, t, :]
        gi = jnp.dot(xt, wih, preferred_element_type=F32) + bih
        gh = jnp.dot(h, whh, preferred_element_type=F32) + bhh
        i_r, i_z, i_n = gi[:, :_AF], gi[:, _AF:2 * _AF], gi[:, 2 * _AF:]
        h_r, h_z, h_n = gh[:, :_AF], gh[:, _AF:2 * _AF], gh[:, 2 * _AF:]
        r = jax.nn.sigmoid(i_r + h_r)
        z = jax.nn.sigmoid(i_z + h_z)
        nn_ = jnp.tanh(i_n + r * h_n)
        h = (1.0 - z) * nn_ + z * h
        out = jnp.where(idx == t, h, out)
    o_ref[...] = out


def _gru(actions_padded, actions_len, rnn):
    w_ih, w_hh, b_ih, b_hh = rnn
    return pl.pallas_call(
        _gru_body,
        out_shape=jax.ShapeDtypeStruct((_C, _AF), F32),
    )(actions_padded, actions_len.reshape(_C, 1), w_ih, w_hh,
      b_ih.reshape(1, 3 * _AF), b_hh.reshape(1, 3 * _AF))


def _pool_body(p_ref, w2_ref, b2_ref, base_ref, o_ref):
    pooled = p_ref[:, :_GW]
    bm0 = pooled.shape[0]
    d = p_ref[:, _GW:_GW + _H * 16].reshape(bm0, _H, 16)[:, :, 0]
    scale = 1.0 / (d + 1e-13)
    ratio = d * scale
    bm = pooled.shape[0]
    ps = (pooled.reshape(bm, _H, _HID) * scale[:, :, None]).reshape(bm, _GW)
    o_ref[...] = (jnp.dot(ps, w2_ref[...], preferred_element_type=F32)
                  + jnp.dot(ratio, b2_ref[...], preferred_element_type=F32)
                  + base_ref[...])


def _pool_update(pooled, w2cat, b2cat, base, bm):
    m = pooled.shape[0]
    return pl.pallas_call(
        _pool_body,
        grid=(m // bm,),
        in_specs=[pl.BlockSpec((bm, _PW), lambda i: (i, 0)),
                  pl.BlockSpec((_GW, _FEA), lambda i: (0, 0)),
                  pl.BlockSpec((_H, _FEA), lambda i: (0, 0)),
                  pl.BlockSpec((bm, _FEA), lambda i: (i, 0))],
        out_specs=pl.BlockSpec((bm, _FEA), lambda i: (i, 0)),
        out_shape=jax.ShapeDtypeStruct((m, _FEA), F32),
    )(pooled, w2cat, b2cat, base)


def _cry_body(x_ref, wg_ref, bg_ref, wm_ref, bm_ref, w2g_ref, b2g_ref,
              pw_ref, o_ref):
    x = x_ref[...]
    bm = x.shape[0]
    g = jnp.dot(x, wg_ref[...], preferred_element_type=F32) + bg_ref[...]
    g = jnp.maximum(g, 0.01 * g)
    logit = jnp.sum(g.reshape(bm, _H, _HID) * w2g_ref[...].reshape(1, _H, _HID),
                    axis=2) + b2g_ref[...]
    q = pw_ref[...] * jnp.exp(logit)
    m_ = jnp.dot(x, wm_ref[...], preferred_element_type=F32) + bm_ref[...]
    m_ = jnp.maximum(m_, 0.01 * m_)
    m_ = (m_.reshape(bm, _H, _HID) * q[:, :, None]).reshape(bm, _GW)
    z15 = jnp.zeros((bm, 15), F32)
    o_ref[...] = jnp.concatenate(
        [m_, q[:, 0:1], z15, q[:, 1:2], z15, q[:, 2:3],
         jnp.zeros((bm, _PW - _GW - 33), F32)], axis=1)


def _cry_tables(prec_fea, wg, bg, wm, bmv, w2g, b2g, pw, bm):
    return pl.pallas_call(
        _cry_body,
        grid=(_N // bm,),
        in_specs=[pl.BlockSpec((bm, _FEA), lambda i: (i, 0)),
                  pl.BlockSpec((_FEA, _GW), lambda i: (0, 0)),
                  pl.BlockSpec((1, _GW), lambda i: (0, 0)),
                  pl.BlockSpec((_FEA, _GW), lambda i: (0, 0)),
                  pl.BlockSpec((1, _GW), lambda i: (0, 0)),
                  pl.BlockSpec((1, _GW), lambda i: (0, 0)),
                  pl.BlockSpec((1, _H), lambda i: (0, 0)),
                  pl.BlockSpec((bm, 1), lambda i: (i, 0))],
        out_specs=pl.BlockSpec((bm, _PW), lambda i: (i, 0)),
        out_shape=jax.ShapeDtypeStruct((_N, _PW), F32),
    )(prec_fea, wg, bg.reshape(1, _GW), wm, bmv.reshape(1, _GW),
      w2g.reshape(1, _GW), b2g.reshape(1, _H), pw)


def _out_body(x_ref, mask_ref, w1, b1, r1, w2, b2, r2, w3, b3, w4, b4, r4,
              wo, bo, o_ref):
    x = x_ref[...]
    h = jnp.maximum(jnp.dot(x, w1[...], preferred_element_type=F32) + b1[...], 0.0)
    x = h + jnp.dot(x, r1[...], preferred_element_type=F32)
    h = jnp.maximum(jnp.dot(x, w2[...], preferred_element_type=F32) + b2[...], 0.0)
    x = h + jnp.dot(x, r2[...], preferred_element_type=F32)
    h = jnp.maximum(jnp.dot(x, w3[...], preferred_element_type=F32) + b3[...], 0.0)
    x = h + x
    h = jnp.maximum(jnp.dot(x, w4[...], preferred_element_type=F32) + b4[...], 0.0)
    x = h + jnp.dot(x, r4[...], preferred_element_type=F32)
    out = jnp.dot(x, wo[...], preferred_element_type=F32) + bo[...]
    o_ref[...] = jnp.where(mask_ref[...] != 0.0, out,
                           jnp.full_like(out, -999.0))


def _out_mlp(react_fea, mask, outp):
    fcs = outp['fcs']
    res = outp['res']
    wo, bo = outp['fc_out']
    t = 81
    args = [react_fea, mask,
            fcs[0][0], fcs[0][1].reshape(1, 256), res[0],
            fcs[1][0], fcs[1][1].reshape(1, 512), res[1],
            fcs[2][0], fcs[2][1].reshape(1, 512),
            fcs[3][0], fcs[3][1].reshape(1, 256), res[3],
            wo, bo.reshape(1, t)]
    return pl.pallas_call(
        _out_body,
        out_shape=jax.ShapeDtypeStruct((_C, t), F32),
    )(*args)


def _scmesh():
    return plsc.VectorSubcoreMesh(core_axis_name="c", subcore_axis_name="s")


def _edge_sc_body(sa_hbm, nb_hbm, act_hbm, nbr2d_hbm, meta_hbm,
                  bnd_hbm, w2_hbm, out_hbm,
                  sa_v, arow_v, nbrs_v, acc_v, gtmp_v, mrow_v, idx_v,
                  bnd_v, w2_v, sem):
    w = lax.axis_index("c") * 16 + lax.axis_index("s")

    pltpu.sync_copy(bnd_hbm.at[w], bnd_v)
    bv = bnd_v[pl.ds(0, 16)]
    n_lo = bv[0]
    n_hi = bv[1]
    pltpu.sync_copy(w2_hbm, w2_v)

    def node_body(n, _):
        pltpu.sync_copy(meta_hbm.at[n], mrow_v)
        mv = mrow_v[pl.ds(0, 16)]
        r = mv[0]
        c0 = mv[1]
        c1 = mv[2]
        e0 = mv[3]
        e1 = mv[4]
        pltpu.sync_copy(sa_hbm.at[n], sa_v)
        pltpu.sync_copy(act_hbm.at[r], arow_v)
        for j in range(_TW // 16):
            sa_v[pl.ds(j * 16, 16)] = (sa_v[pl.ds(j * 16, 16)]
                                       + arow_v[pl.ds(j * 16, 16)])
        z16 = jnp.zeros((16,), F32)
        for j in range(_PW // 16):
            acc_v[pl.ds(j * 16, 16)] = z16

        def chunk_body(c, _c):
            pltpu.sync_copy(nbr2d_hbm.at[c], idx_v)
            ids = idx_v[pl.ds(0, 16)]
            pltpu.async_copy(nb_hbm.at[ids], nbrs_v, sem).wait()

            def lane_body(e, _l):
                eid = c * 16 + e

                @pl.when((eid >= e0) & (eid < e1))
                def _valid():
                    wv = nbrs_v[e, pl.ds(_TW, 16)]
                    b2v = w2_v[_GW // 16, pl.ds(0, 16)]
                    for h in range(_H):
                        gacc = jnp.zeros((16,), F32)
                        for j in range(_HID // 16):
                            off = h * _HID + j * 16
                            x = sa_v[pl.ds(off, 16)] + nbrs_v[e, pl.ds(off, 16)]
                            x = jnp.maximum(x, 0.01 * x)
                            gacc = gacc + x * w2_v[h * 16 + j, pl.ds(0, 16)]
                        gtmp_v[pl.ds(0, 16)] = gacc
                        gv = gtmp_v[pl.ds(0, 16)]
                        gl = b2v[h]
                        for t in range(16):
                            gl = gl + gv[t]
                        qvec = wv * jnp.exp(jnp.full((16,), gl, F32))
                        doff = _GW + 16 * h
                        acc_v[pl.ds(doff, 16)] = acc_v[pl.ds(doff, 16)] + qvec
                        gtmp_v[pl.ds(0, 16)] = qvec
                        qv2 = gtmp_v[pl.ds(0, 16)]
                        q = qv2[0]
                        for j in range(_HID // 16):
                            off = _GW + h * _HID + j * 16
                            x = sa_v[pl.ds(off, 16)] + nbrs_v[e, pl.ds(off, 16)]
                            x = jnp.maximum(x, 0.01 * x)
                            aoff = h * _HID + j * 16
                            acc_v[pl.ds(aoff, 16)] = (acc_v[pl.ds(aoff, 16)]
                                                      + x * q)
                return _l

            lax.fori_loop(0, 16, lane_body, 0)
            return _c

        lax.fori_loop(c0, c1, chunk_body, 0)
        pltpu.sync_copy(acc_v, out_hbm.at[n])
        return _

    lax.fori_loop(n_lo, n_hi, node_body, 0)


def _edge_attention(sa_all, nb_all, act_all, nbr2d, meta, bnd, w2tab):
    k = functools.partial(
        pl.kernel, mesh=_scmesh(),
        out_type=jax.ShapeDtypeStruct((_N, _PW), F32),
        scratch_types=[pltpu.VMEM((_TW,), F32),
                       pltpu.VMEM((_TW,), F32),
                       pltpu.VMEM((16, _NBW), F32),
                       pltpu.VMEM((_PW,), F32),
                       pltpu.VMEM((16,), F32),
                       pltpu.VMEM((16,), I32),
                       pltpu.VMEM((16,), I32),
                       pltpu.VMEM((16,), I32),
                       pltpu.VMEM((_GW // 16 + 1, 16), F32),
                       pltpu.SemaphoreType.DMA],
    )(_edge_sc_body)
    return k(sa_all, nb_all, act_all, nbr2d, meta, bnd, w2tab)


def _segsum_sc_body(t_hbm, idx2d_hbm, meta_hbm, bnd_hbm, out_hbm,
                    rows_v, acc_v, mrow_v, idx_v, bnd_v, sem):
    w = lax.axis_index("c") * 16 + lax.axis_index("s")

    pltpu.sync_copy(bnd_hbm.at[w], bnd_v)
    bv = bnd_v[pl.ds(0, 16)]
    n_lo = bv[0]
    n_hi = bv[1]

    def seg_body(n, _):
        pltpu.sync_copy(meta_hbm.at[n], mrow_v)
        mv = mrow_v[pl.ds(0, 16)]
        c0 = mv[0]
        c1 = mv[1]
        e0 = mv[2]
        e1 = mv[3]
        z16 = jnp.zeros((16,), F32)
        for j in range(_PW // 16):
            acc_v[pl.ds(j * 16, 16)] = z16

        def chunk_body(c, _c):
            pltpu.sync_copy(idx2d_hbm.at[c], idx_v)
            ids = idx_v[pl.ds(0, 16)]
            pltpu.async_copy(t_hbm.at[ids], rows_v, sem).wait()

            def lane_body(e, _l):
                eid = c * 16 + e

                @pl.when((eid >= e0) & (eid < e1))
                def _valid():
                    for j in range(_PW // 16):
                        acc_v[pl.ds(j * 16, 16)] = (acc_v[pl.ds(j * 16, 16)]
                                                    + rows_v[e, pl.ds(j * 16, 16)])
                return _l

            lax.fori_loop(0, 16, lane_body, 0)
            return _c

        lax.fori_loop(c0, c1, chunk_body, 0)
        pltpu.sync_copy(acc_v, out_hbm.at[n])
        return _

    lax.fori_loop(n_lo, n_hi, seg_body, 0)


def _segsum(tbl, idx2d, meta, bnd):
    k = functools.partial(
        pl.kernel, mesh=_scmesh(),
        out_type=jax.ShapeDtypeStruct((_C, _PW), F32),
        scratch_types=[pltpu.VMEM((16, _PW), F32),
                       pltpu.VMEM((_PW,), F32),
                       pltpu.VMEM((16,), I32),
                       pltpu.VMEM((16,), I32),
                       pltpu.VMEM((16,), I32),
                       pltpu.SemaphoreType.DMA],
    )(_segsum_sc_body)
    return k(tbl, idx2d, meta, bnd)


def _seg_meta(idx_sorted, nseg, nitems):
    off = jnp.searchsorted(idx_sorted, jnp.arange(nseg + 1), side='left')
    off = off.astype(I32)
    e0 = off[:-1]
    e1 = off[1:]
    c0 = e0 // 16
    c1 = jnp.where(e1 > e0, (e1 + 15) // 16, c0)
    return off, c0, c1


def _worker_bnd(off, nseg):
    total = off[-1]
    tgt = (jnp.arange(_NWORK + 1) * total) // _NWORK
    bnd = jnp.searchsorted(off, tgt, side='left').astype(I32)
    bnd = bnd.at[0].set(0).at[_NWORK].set(nseg)
    rows = jnp.stack([bnd[:-1], bnd[1:]], axis=1)
    return jnp.pad(rows, ((0, 0), (0, 14)))


def _pack16(x, pad_val, rows):
    need = rows * 16 - x.shape[0]
    return jnp.pad(x, (0, need), constant_values=pad_val).reshape(rows, 16)


def kernel(prec_weights, orig_prec_fea, self_fea_idx, nbr_fea_idx,
           reaction_prec_idx, actions_padded, actions_len, prec_elem_mask,
           params):
    self_fea_idx = self_fea_idx.astype(I32)
    nbr_fea_idx = nbr_fea_idx.astype(I32)
    reaction_prec_idx = reaction_prec_idx.astype(I32)

    actions = _gru(actions_padded, actions_len.astype(I32), params['rnn'])

    emb = _mm(orig_prec_fea, params['W_emb'],
              jnp.zeros((_FEA - 1,), F32), 1000)
    prec_fea = jnp.concatenate([emb, prec_weights], axis=1)

    off, c0, c1 = _seg_meta(self_fea_idx, _N, _E)
    meta = jnp.stack([reaction_prec_idx, c0, c1, off[:-1], off[1:]], axis=1)
    meta = jnp.pad(meta, ((0, 0), (0, 11)))
    bnd = _worker_bnd(off, _N)
    nbr2d = _pack16(nbr_fea_idx, 0, _ECH)

    for heads in params['graphs']:
        ws_l, wn_l, wa_l, bs_l = [], [], [], []
        w2g_l, b2g_l, w2m_l, b2m_l = [], [], [], []
        for net in ('gate', 'msg'):
            for h in heads:
                (w1, b1), (w2, b2) = h[net]
                ws_l.append(w1[:_FEA])
                wn_l.append(w1[_FEA:2 * _FEA])
                wa_l.append(w1[2 * _FEA:])
                bs_l.append(b1)
                if net == 'gate':
                    w2g_l.append(w2[:, 0])
                    b2g_l.append(b2[0])
                else:
                    w2m_l.append(w2)
                    b2m_l.append(b2)
        ws = jnp.concatenate(ws_l, axis=1)
        wn = jnp.concatenate(wn_l, axis=1)
        wa = jnp.concatenate(wa_l, axis=1)
        bs = jnp.concatenate(bs_l)
        sa_all = _mm(prec_fea, ws, bs, 1000)
        nb_all = _mm(prec_fea, wn, jnp.zeros((_TW,), F32), 1000)
        nb_all = jnp.concatenate(
            [nb_all, prec_weights, jnp.zeros((_N, 127), F32)], axis=1)
        act_all = _mm(actions, wa, jnp.zeros((_TW,), F32), 1000)
        w2tab = jnp.concatenate(
            [jnp.concatenate(w2g_l).reshape(_GW // 16, 16),
             jnp.pad(jnp.stack(b2g_l), (0, 13)).reshape(1, 16)], axis=0)
        pooled = _edge_attention(sa_all, nb_all, act_all, nbr2d,
                                 meta, bnd, w2tab)
        w2cat = jnp.concatenate([w / _H for w in w2m_l], axis=0)
        b2cat = jnp.stack([b / _H for b in b2m_l], axis=0)
        prec_fea = _pool_update(pooled, w2cat, b2cat, prec_fea, 1000)

    wg_l, bg_l, wm_l, bm_l = [], [], [], []
    w2g_l, b2g_l, w2m_l, b2m_l = [], [], [], []
    for h in params['cry_pool']:
        (w1, b1), (w2, b2) = h['gate']
        wg_l.append(w1)
        bg_l.append(b1)
        w2g_l.append(w2[:, 0])
        b2g_l.append(b2[0])
        (w1, b1), (w2, b2) = h['msg']
        wm_l.append(w1)
        bm_l.append(b1)
        w2m_l.append(w2)
        b2m_l.append(b2)
    tbl = _cry_tables(prec_fea,
                      jnp.concatenate(wg_l, axis=1), jnp.concatenate(bg_l),
                      jnp.concatenate(wm_l, axis=1), jnp.concatenate(bm_l),
                      jnp.concatenate(w2g_l), jnp.stack(b2g_l),
                      prec_weights, 1000)

    roff, rc0, rc1 = _seg_meta(reaction_prec_idx, _C, _N)
    rmeta = jnp.pad(jnp.stack([rc0, rc1, roff[:-1], roff[1:]], axis=1),
                    ((0, 0), (0, 12)))
    rbnd = _worker_bnd(roff, _C)
    idx2d = jnp.arange(_N, dtype=I32).reshape(_NCH, 16)
    pooled2 = _segsum(tbl, idx2d, rmeta, rbnd)

    w2cat = jnp.concatenate([w / _H for w in w2m_l], axis=0)
    b2cat = jnp.stack([b / _H for b in b2m_l], axis=0)
    react_fea = _pool_update(pooled2, w2cat, b2cat,
                             jnp.zeros((_C, _FEA), F32), 1000)

    output = _out_mlp(react_fea, prec_elem_mask, params['out'])
    return output, react_fea


# speedup vs baseline: 1.6558x; 1.0025x over previous
import functools

import jax
import jax.numpy as jnp
from jax import lax
from jax.experimental import pallas as pl
from jax.experimental.pallas import tpu as pltpu
from jax.experimental.pallas import tpu_sc as plsc

F32 = jnp.float32
I32 = jnp.int32

_N = 10000
_E = 320000
_C = 2000
_FEA = 128
_AF = 32
_H = 3
_HID = 256
_GW = _H * _HID
_TW = 2 * _GW
_NBW = _TW + 128
_PW = _GW + 128
_ECH = _E // 16
_NCH = _N // 16
_NWORK = 32


def _mm_body(x_ref, w_ref, b_ref, o_ref):
    o_ref[...] = jnp.dot(x_ref[...], w_ref[...],
                         preferred_element_type=F32) + b_ref[...]


def _mm(x, w, b, bm):
    m, k = x.shape
    n = w.shape[1]
    grid = m // bm
    return pl.pallas_call(
        _mm_body,
        grid=(grid,),
        in_specs=[pl.BlockSpec((bm, k), lambda i: (i, 0)),
                  pl.BlockSpec((k, n), lambda i: (0, 0)),
                  pl.BlockSpec((1, n), lambda i: (0, 0))],
        out_specs=pl.BlockSpec((bm, n), lambda i: (i, 0)),
        out_shape=jax.ShapeDtypeStruct((m, n), F32),
    )(x, w, b.reshape(1, n))


def _gru_body(x_ref, len_ref, wih_ref, whh_ref, bih_ref, bhh_ref, o_ref):
    cdim = x_ref.shape[0]
    h = jnp.zeros((cdim, _AF), F32)
    out = jnp.zeros((cdim, _AF), F32)
    idx = jnp.clip(len_ref[...] - 1, 0, 9)
    wih = wih_ref[...]
    whh = whh_ref[...]
    bih = bih_ref[...]
    bhh = bhh_ref[...]
    for t in range(10):
        xt = x_ref[:, t, :]
        gi = jnp.dot(xt, wih, preferred_element_type=F32) + bih
        gh = jnp.dot(h, whh, preferred_element_type=F32) + bhh
        i_r, i_z, i_n = gi[:, :_AF], gi[:, _AF:2 * _AF], gi[:, 2 * _AF:]
        h_r, h_z, h_n = gh[:, :_AF], gh[:, _AF:2 * _AF], gh[:, 2 * _AF:]
        r = jax.nn.sigmoid(i_r + h_r)
        z = jax.nn.sigmoid(i_z + h_z)
        nn_ = jnp.tanh(i_n + r * h_n)
        h = (1.0 - z) * nn_ + z * h
        out = jnp.where(idx == t, h, out)
    o_ref[...] = out


def _gru(actions_padded, actions_len, rnn):
    w_ih, w_hh, b_ih, b_hh = rnn
    return pl.pallas_call(
        _gru_body,
        out_shape=jax.ShapeDtypeStruct((_C, _AF), F32),
    )(actions_padded, actions_len.reshape(_C, 1), w_ih, w_hh,
      b_ih.reshape(1, 3 * _AF), b_hh.reshape(1, 3 * _AF))


def _pool_body(p_ref, w2_ref, b2_ref, base_ref, o_ref):
    pooled = p_ref[:, :_GW]
    bm0 = pooled.shape[0]
    d = p_ref[:, _GW:_GW + _H * 16].reshape(bm0, _H, 16)[:, :, 0]
    scale = 1.0 / (d + 1e-13)
    ratio = d * scale
    bm = pooled.shape[0]
    ps = (pooled.reshape(bm, _H, _HID) * scale[:, :, None]).reshape(bm, _GW)
    o_ref[...] = (jnp.dot(ps, w2_ref[...], preferred_element_type=F32)
                  + jnp.dot(ratio, b2_ref[...], preferred_element_type=F32)
                  + base_ref[...])


def _pool_update(pooled, w2cat, b2cat, base, bm):
    m = pooled.shape[0]
    return pl.pallas_call(
        _pool_body,
        grid=(m // bm,),
        in_specs=[pl.BlockSpec((bm, _PW), lambda i: (i, 0)),
                  pl.BlockSpec((_GW, _FEA), lambda i: (0, 0)),
                  pl.BlockSpec((_H, _FEA), lambda i: (0, 0)),
                  pl.BlockSpec((bm, _FEA), lambda i: (i, 0))],
        out_specs=pl.BlockSpec((bm, _FEA), lambda i: (i, 0)),
        out_shape=jax.ShapeDtypeStruct((m, _FEA), F32),
    )(pooled, w2cat, b2cat, base)


def _cry_body(x_ref, wg_ref, bg_ref, wm_ref, bm_ref, w2g_ref, b2g_ref,
              pw_ref, o_ref):
    x = x_ref[...]
    bm = x.shape[0]
    g = jnp.dot(x, wg_ref[...], preferred_element_type=F32) + bg_ref[...]
    g = jnp.maximum(g, 0.01 * g)
    logit = jnp.sum(g.reshape(bm, _H, _HID) * w2g_ref[...].reshape(1, _H, _HID),
                    axis=2) + b2g_ref[...]
    q = pw_ref[...] * jnp.exp(logit)
    m_ = jnp.dot(x, wm_ref[...], preferred_element_type=F32) + bm_ref[...]
    m_ = jnp.maximum(m_, 0.01 * m_)
    m_ = (m_.reshape(bm, _H, _HID) * q[:, :, None]).reshape(bm, _GW)
    z15 = jnp.zeros((bm, 15), F32)
    o_ref[...] = jnp.concatenate(
        [m_, q[:, 0:1], z15, q[:, 1:2], z15, q[:, 2:3],
         jnp.zeros((bm, _PW - _GW - 33), F32)], axis=1)


def _cry_tables(prec_fea, wg, bg, wm, bmv, w2g, b2g, pw, bm):
    return pl.pallas_call(
        _cry_body,
        grid=(_N // bm,),
        in_specs=[pl.BlockSpec((bm, _FEA), lambda i: (i, 0)),
                  pl.BlockSpec((_FEA, _GW), lambda i: (0, 0)),
                  pl.BlockSpec((1, _GW), lambda i: (0, 0)),
                  pl.BlockSpec((_FEA, _GW), lambda i: (0, 0)),
                  pl.BlockSpec((1, _GW), lambda i: (0, 0)),
                  pl.BlockSpec((1, _GW), lambda i: (0, 0)),
                  pl.BlockSpec((1, _H), lambda i: (0, 0)),
                  pl.BlockSpec((bm, 1), lambda i: (i, 0))],
        out_specs=pl.BlockSpec((bm, _PW), lambda i: (i, 0)),
        out_shape=jax.ShapeDtypeStruct((_N, _PW), F32),
    )(prec_fea, wg, bg.reshape(1, _GW), wm, bmv.reshape(1, _GW),
      w2g.reshape(1, _GW), b2g.reshape(1, _H), pw)


def _out_body(x_ref, mask_ref, w1, b1, r1, w2, b2, r2, w3, b3, w4, b4, r4,
              wo, bo, o_ref):
    x = x_ref[...]
    h = jnp.maximum(jnp.dot(x, w1[...], preferred_element_type=F32) + b1[...], 0.0)
    x = h + jnp.dot(x, r1[...], preferred_element_type=F32)
    h = jnp.maximum(jnp.dot(x, w2[...], preferred_element_type=F32) + b2[...], 0.0)
    x = h + jnp.dot(x, r2[...], preferred_element_type=F32)
    h = jnp.maximum(jnp.dot(x, w3[...], preferred_element_type=F32) + b3[...], 0.0)
    x = h + x
    h = jnp.maximum(jnp.dot(x, w4[...], preferred_element_type=F32) + b4[...], 0.0)
    x = h + jnp.dot(x, r4[...], preferred_element_type=F32)
    out = jnp.dot(x, wo[...], preferred_element_type=F32) + bo[...]
    o_ref[...] = jnp.where(mask_ref[...] != 0.0, out,
                           jnp.full_like(out, -999.0))


def _out_mlp(react_fea, mask, outp):
    fcs = outp['fcs']
    res = outp['res']
    wo, bo = outp['fc_out']
    t = 81
    args = [react_fea, mask,
            fcs[0][0], fcs[0][1].reshape(1, 256), res[0],
            fcs[1][0], fcs[1][1].reshape(1, 512), res[1],
            fcs[2][0], fcs[2][1].reshape(1, 512),
            fcs[3][0], fcs[3][1].reshape(1, 256), res[3],
            wo, bo.reshape(1, t)]
    return pl.pallas_call(
        _out_body,
        out_shape=jax.ShapeDtypeStruct((_C, t), F32),
    )(*args)


def _scmesh():
    return plsc.VectorSubcoreMesh(core_axis_name="c", subcore_axis_name="s")


def _edge_sc_body(sa_hbm, nb_hbm, act_hbm, nbr2d_hbm, meta_hbm,
                  bnd_hbm, w2_hbm, out_hbm,
                  sa_v, arow_v, nbrs_v, acc_v, gtmp_v, mrow_v, idx_v,
                  bnd_v, w2_v, sem):
    w = lax.axis_index("c") * 16 + lax.axis_index("s")

    pltpu.sync_copy(bnd_hbm.at[w], bnd_v)
    bv = bnd_v[pl.ds(0, 16)]
    n_lo = bv[0]
    n_hi = bv[1]
    pltpu.sync_copy(w2_hbm, w2_v)

    def node_body(n, _):
        pltpu.sync_copy(meta_hbm.at[n], mrow_v)
        mv = mrow_v[pl.ds(0, 16)]
        r = mv[0]
        c0 = mv[1]
        c1 = mv[2]
        e0 = mv[3]
        e1 = mv[4]
        pltpu.sync_copy(sa_hbm.at[n], sa_v)
        pltpu.sync_copy(act_hbm.at[r], arow_v)
        for j in range(_TW // 16):
            sa_v[pl.ds(j * 16, 16)] = (sa_v[pl.ds(j * 16, 16)]
                                       + arow_v[pl.ds(j * 16, 16)])
        z16 = jnp.zeros((16,), F32)
        for j in range(_PW // 16):
            acc_v[pl.ds(j * 16, 16)] = z16

        def chunk_body(c, _c):
            pltpu.sync_copy(nbr2d_hbm.at[c], idx_v)
            ids = idx_v[pl.ds(0, 16)]
            pltpu.async_copy(nb_hbm.at[ids], nbrs_v, sem).wait()

            b2v_c = w2_v[_GW // 16, pl.ds(0, 16)]

            def lane_body(e, _l):
                eid = c * 16 + e

                @pl.when((eid >= e0) & (eid < e1))
                def _valid():
                    wv = nbrs_v[e, pl.ds(_TW, 16)]
                    for h in range(_H):
                        gacc = jnp.zeros((16,), F32)
                        for j in range(_HID // 16):
                            off = h * _HID + j * 16
                            x = sa_v[pl.ds(off, 16)] + nbrs_v[e, pl.ds(off, 16)]
                            x = jnp.maximum(x, 0.01 * x)
                            gacc = gacc + x * w2_v[h * 16 + j, pl.ds(0, 16)]
                        gtmp_v[pl.ds(0, 16)] = gacc
                        gv = gtmp_v[pl.ds(0, 16)]
                        s0 = (gv[0] + gv[1]) + (gv[2] + gv[3])
                        s1 = (gv[4] + gv[5]) + (gv[6] + gv[7])
                        s2 = (gv[8] + gv[9]) + (gv[10] + gv[11])
                        s3 = (gv[12] + gv[13]) + (gv[14] + gv[15])
                        gl = (s0 + s1) + (s2 + s3) + b2v_c[h]
                        qvec = wv * jnp.exp(jnp.full((16,), gl, F32))
                        doff = _GW + 16 * h
                        acc_v[pl.ds(doff, 16)] = acc_v[pl.ds(doff, 16)] + qvec
                        gtmp_v[pl.ds(0, 16)] = qvec
                        qv2 = gtmp_v[pl.ds(0, 16)]
                        q = qv2[0]
                        for j in range(_HID // 16):
                            off = _GW + h * _HID + j * 16
                            x = sa_v[pl.ds(off, 16)] + nbrs_v[e, pl.ds(off, 16)]
                            x = jnp.maximum(x, 0.01 * x)
                            aoff = h * _HID + j * 16
                            acc_v[pl.ds(aoff, 16)] = (acc_v[pl.ds(aoff, 16)]
                                                      + x * q)
                return _l

            lax.fori_loop(0, 16, lane_body, 0)
            return _c

        lax.fori_loop(c0, c1, chunk_body, 0)
        pltpu.sync_copy(acc_v, out_hbm.at[n])
        return _

    lax.fori_loop(n_lo, n_hi, node_body, 0)


def _edge_attention(sa_all, nb_all, act_all, nbr2d, meta, bnd, w2tab):
    k = functools.partial(
        pl.kernel, mesh=_scmesh(),
        out_type=jax.ShapeDtypeStruct((_N, _PW), F32),
        scratch_types=[pltpu.VMEM((_TW,), F32),
                       pltpu.VMEM((_TW,), F32),
                       pltpu.VMEM((16, _NBW), F32),
                       pltpu.VMEM((_PW,), F32),
                       pltpu.VMEM((16,), F32),
                       pltpu.VMEM((16,), I32),
                       pltpu.VMEM((16,), I32),
                       pltpu.VMEM((16,), I32),
                       pltpu.VMEM((_GW // 16 + 1, 16), F32),
                       pltpu.SemaphoreType.DMA],
    )(_edge_sc_body)
    return k(sa_all, nb_all, act_all, nbr2d, meta, bnd, w2tab)


def _segsum_sc_body(t_hbm, idx2d_hbm, meta_hbm, bnd_hbm, out_hbm,
                    rows_v, acc_v, mrow_v, idx_v, bnd_v, sem):
    w = lax.axis_index("c") * 16 + lax.axis_index("s")

    pltpu.sync_copy(bnd_hbm.at[w], bnd_v)
    bv = bnd_v[pl.ds(0, 16)]
    n_lo = bv[0]
    n_hi = bv[1]

    def seg_body(n, _):
        pltpu.sync_copy(meta_hbm.at[n], mrow_v)
        mv = mrow_v[pl.ds(0, 16)]
        c0 = mv[0]
        c1 = mv[1]
        e0 = mv[2]
        e1 = mv[3]
        z16 = jnp.zeros((16,), F32)
        for j in range(_PW // 16):
            acc_v[pl.ds(j * 16, 16)] = z16

        def chunk_body(c, _c):
            pltpu.sync_copy(idx2d_hbm.at[c], idx_v)
            ids = idx_v[pl.ds(0, 16)]
            pltpu.async_copy(t_hbm.at[ids], rows_v, sem).wait()

            def lane_body(e, _l):
                eid = c * 16 + e

                @pl.when((eid >= e0) & (eid < e1))
                def _valid():
                    for j in range(_PW // 16):
                        acc_v[pl.ds(j * 16, 16)] = (acc_v[pl.ds(j * 16, 16)]
                                                    + rows_v[e, pl.ds(j * 16, 16)])
                return _l

            lax.fori_loop(0, 16, lane_body, 0)
            return _c

        lax.fori_loop(c0, c1, chunk_body, 0)
        pltpu.sync_copy(acc_v, out_hbm.at[n])
        return _

    lax.fori_loop(n_lo, n_hi, seg_body, 0)


def _segsum(tbl, idx2d, meta, bnd):
    k = functools.partial(
        pl.kernel, mesh=_scmesh(),
        out_type=jax.ShapeDtypeStruct((_C, _PW), F32),
        scratch_types=[pltpu.VMEM((16, _PW), F32),
                       pltpu.VMEM((_PW,), F32),
                       pltpu.VMEM((16,), I32),
                       pltpu.VMEM((16,), I32),
                       pltpu.VMEM((16,), I32),
                       pltpu.SemaphoreType.DMA],
    )(_segsum_sc_body)
    return k(tbl, idx2d, meta, bnd)


def _seg_meta(idx_sorted, nseg, nitems):
    off = jnp.searchsorted(idx_sorted, jnp.arange(nseg + 1), side='left')
    off = off.astype(I32)
    e0 = off[:-1]
    e1 = off[1:]
    c0 = e0 // 16
    c1 = jnp.where(e1 > e0, (e1 + 15) // 16, c0)
    return off, c0, c1


def _worker_bnd(off, nseg):
    total = off[-1]
    tgt = (jnp.arange(_NWORK + 1) * total) // _NWORK
    bnd = jnp.searchsorted(off, tgt, side='left').astype(I32)
    bnd = bnd.at[0].set(0).at[_NWORK].set(nseg)
    rows = jnp.stack([bnd[:-1], bnd[1:]], axis=1)
    return jnp.pad(rows, ((0, 0), (0, 14)))


def _pack16(x, pad_val, rows):
    need = rows * 16 - x.shape[0]
    return jnp.pad(x, (0, need), constant_values=pad_val).reshape(rows, 16)


def kernel(prec_weights, orig_prec_fea, self_fea_idx, nbr_fea_idx,
           reaction_prec_idx, actions_padded, actions_len, prec_elem_mask,
           params):
    self_fea_idx = self_fea_idx.astype(I32)
    nbr_fea_idx = nbr_fea_idx.astype(I32)
    reaction_prec_idx = reaction_prec_idx.astype(I32)

    actions = _gru(actions_padded, actions_len.astype(I32), params['rnn'])

    emb = _mm(orig_prec_fea, params['W_emb'],
              jnp.zeros((_FEA - 1,), F32), 1000)
    prec_fea = jnp.concatenate([emb, prec_weights], axis=1)

    off, c0, c1 = _seg_meta(self_fea_idx, _N, _E)
    meta = jnp.stack([reaction_prec_idx, c0, c1, off[:-1], off[1:]], axis=1)
    meta = jnp.pad(meta, ((0, 0), (0, 11)))
    bnd = _worker_bnd(off, _N)
    nbr2d = _pack16(nbr_fea_idx, 0, _ECH)

    for heads in params['graphs']:
        ws_l, wn_l, wa_l, bs_l = [], [], [], []
        w2g_l, b2g_l, w2m_l, b2m_l = [], [], [], []
        for net in ('gate', 'msg'):
            for h in heads:
                (w1, b1), (w2, b2) = h[net]
                ws_l.append(w1[:_FEA])
                wn_l.append(w1[_FEA:2 * _FEA])
                wa_l.append(w1[2 * _FEA:])
                bs_l.append(b1)
                if net == 'gate':
                    w2g_l.append(w2[:, 0])
                    b2g_l.append(b2[0])
                else:
                    w2m_l.append(w2)
                    b2m_l.append(b2)
        ws = jnp.concatenate(ws_l, axis=1)
        wn = jnp.concatenate(wn_l, axis=1)
        wa = jnp.concatenate(wa_l, axis=1)
        bs = jnp.concatenate(bs_l)
        both = _mm(prec_fea, jnp.concatenate([ws, wn], axis=1),
                   jnp.concatenate([bs, jnp.zeros((_TW,), F32)]), 1000)
        sa_all = both[:, :_TW]
        nb_all = jnp.concatenate(
            [both[:, _TW:], prec_weights, jnp.zeros((_N, 127), F32)], axis=1)
        act_all = _mm(actions, wa, jnp.zeros((_TW,), F32), 1000)
        w2tab = jnp.concatenate(
            [jnp.concatenate(w2g_l).reshape(_GW // 16, 16),
             jnp.pad(jnp.stack(b2g_l), (0, 13)).reshape(1, 16)], axis=0)
        pooled = _edge_attention(sa_all, nb_all, act_all, nbr2d,
                                 meta, bnd, w2tab)
        w2cat = jnp.concatenate([w / _H for w in w2m_l], axis=0)
        b2cat = jnp.stack([b / _H for b in b2m_l], axis=0)
        prec_fea = _pool_update(pooled, w2cat, b2cat, prec_fea, 1000)

    wg_l, bg_l, wm_l, bm_l = [], [], [], []
    w2g_l, b2g_l, w2m_l, b2m_l = [], [], [], []
    for h in params['cry_pool']:
        (w1, b1), (w2, b2) = h['gate']
        wg_l.append(w1)
        bg_l.append(b1)
        w2g_l.append(w2[:, 0])
        b2g_l.append(b2[0])
        (w1, b1), (w2, b2) = h['msg']
        wm_l.append(w1)
        bm_l.append(b1)
        w2m_l.append(w2)
        b2m_l.append(b2)
    tbl = _cry_tables(prec_fea,
                      jnp.concatenate(wg_l, axis=1), jnp.concatenate(bg_l),
                      jnp.concatenate(wm_l, axis=1), jnp.concatenate(bm_l),
                      jnp.concatenate(w2g_l), jnp.stack(b2g_l),
                      prec_weights, 1000)

    roff, rc0, rc1 = _seg_meta(reaction_prec_idx, _C, _N)
    rmeta = jnp.pad(jnp.stack([rc0, rc1, roff[:-1], roff[1:]], axis=1),
                    ((0, 0), (0, 12)))
    rbnd = _worker_bnd(roff, _C)
    idx2d = jnp.arange(_N, dtype=I32).reshape(_NCH, 16)
    pooled2 = _segsum(tbl, idx2d, rmeta, rbnd)

    w2cat = jnp.concatenate([w / _H for w in w2m_l], axis=0)
    b2cat = jnp.stack([b / _H for b in b2m_l], axis=0)
    react_fea = _pool_update(pooled2, w2cat, b2cat,
                             jnp.zeros((_C, _FEA), F32), 1000)

    output = _out_mlp(react_fea, prec_elem_mask, params['out'])
    return output, react_fea


# speedup vs baseline: 1.6825x; 1.0161x over previous
import functools

import jax
import jax.numpy as jnp
from jax import lax
from jax.experimental import pallas as pl
from jax.experimental.pallas import tpu as pltpu
from jax.experimental.pallas import tpu_sc as plsc

F32 = jnp.float32
I32 = jnp.int32

_N = 10000
_E = 320000
_C = 2000
_FEA = 128
_AF = 32
_H = 3
_HID = 256
_GW = _H * _HID
_TW = 2 * _GW
_NBW = _TW + 128
_PW = _GW + 128
_ECH = _E // 16
_NCH = _N // 16
_NWORK = 32


def _mm_body(x_ref, w_ref, b_ref, o_ref):
    o_ref[...] = jnp.dot(x_ref[...], w_ref[...],
                         preferred_element_type=F32) + b_ref[...]


def _mm(x, w, b, bm):
    m, k = x.shape
    n = w.shape[1]
    grid = m // bm
    return pl.pallas_call(
        _mm_body,
        grid=(grid,),
        in_specs=[pl.BlockSpec((bm, k), lambda i: (i, 0)),
                  pl.BlockSpec((k, n), lambda i: (0, 0)),
                  pl.BlockSpec((1, n), lambda i: (0, 0))],
        out_specs=pl.BlockSpec((bm, n), lambda i: (i, 0)),
        out_shape=jax.ShapeDtypeStruct((m, n), F32),
    )(x, w, b.reshape(1, n))


def _gru_body(x_ref, len_ref, wih_ref, whh_ref, bih_ref, bhh_ref, o_ref):
    cdim = x_ref.shape[0]
    h = jnp.zeros((cdim, _AF), F32)
    out = jnp.zeros((cdim, _AF), F32)
    idx = jnp.clip(len_ref[...] - 1, 0, 9)
    wih = wih_ref[...]
    whh = whh_ref[...]
    bih = bih_ref[...]
    bhh = bhh_ref[...]
    for t in range(10):
        xt = x_ref[:, t, :]
        gi = jnp.dot(xt, wih, preferred_element_type=F32) + bih
        gh = jnp.dot(h, whh, preferred_element_type=F32) + bhh
        i_r, i_z, i_n = gi[:, :_AF], gi[:, _AF:2 * _AF], gi[:, 2 * _AF:]
        h_r, h_z, h_n = gh[:, :_AF], gh[:, _AF:2 * _AF], gh[:, 2 * _AF:]
        r = jax.nn.sigmoid(i_r + h_r)
        z = jax.nn.sigmoid(i_z + h_z)
        nn_ = jnp.tanh(i_n + r * h_n)
        h = (1.0 - z) * nn_ + z * h
        out = jnp.where(idx == t, h, out)
    o_ref[...] = out


def _gru(actions_padded, actions_len, rnn):
    w_ih, w_hh, b_ih, b_hh = rnn
    return pl.pallas_call(
        _gru_body,
        out_shape=jax.ShapeDtypeStruct((_C, _AF), F32),
    )(actions_padded, actions_len.reshape(_C, 1), w_ih, w_hh,
      b_ih.reshape(1, 3 * _AF), b_hh.reshape(1, 3 * _AF))


def _pool_body(p_ref, w2_ref, b2_ref, base_ref, o_ref):
    pooled = p_ref[:, :_GW]
    bm0 = pooled.shape[0]
    d = p_ref[:, _GW:_GW + _H * 16].reshape(bm0, _H, 16)[:, :, 0]
    scale = 1.0 / (d + 1e-13)
    ratio = d * scale
    bm = pooled.shape[0]
    ps = (pooled.reshape(bm, _H, _HID) * scale[:, :, None]).reshape(bm, _GW)
    o_ref[...] = (jnp.dot(ps, w2_ref[...], preferred_element_type=F32)
                  + jnp.dot(ratio, b2_ref[...], preferred_element_type=F32)
                  + base_ref[...])


def _pool_update(pooled, w2cat, b2cat, base, bm):
    m = pooled.shape[0]
    return pl.pallas_call(
        _pool_body,
        grid=(m // bm,),
        in_specs=[pl.BlockSpec((bm, _PW), lambda i: (i, 0)),
                  pl.BlockSpec((_GW, _FEA), lambda i: (0, 0)),
                  pl.BlockSpec((_H, _FEA), lambda i: (0, 0)),
                  pl.BlockSpec((bm, _FEA), lambda i: (i, 0))],
        out_specs=pl.BlockSpec((bm, _FEA), lambda i: (i, 0)),
        out_shape=jax.ShapeDtypeStruct((m, _FEA), F32),
    )(pooled, w2cat, b2cat, base)


def _cry_body(x_ref, wg_ref, bg_ref, wm_ref, bm_ref, w2g_ref, b2g_ref,
              pw_ref, o_ref):
    x = x_ref[...]
    bm = x.shape[0]
    g = jnp.dot(x, wg_ref[...], preferred_element_type=F32) + bg_ref[...]
    g = jnp.maximum(g, 0.01 * g)
    logit = jnp.sum(g.reshape(bm, _H, _HID) * w2g_ref[...].reshape(1, _H, _HID),
                    axis=2) + b2g_ref[...]
    q = pw_ref[...] * jnp.exp(logit)
    m_ = jnp.dot(x, wm_ref[...], preferred_element_type=F32) + bm_ref[...]
    m_ = jnp.maximum(m_, 0.01 * m_)
    m_ = (m_.reshape(bm, _H, _HID) * q[:, :, None]).reshape(bm, _GW)
    z15 = jnp.zeros((bm, 15), F32)
    o_ref[...] = jnp.concatenate(
        [m_, q[:, 0:1], z15, q[:, 1:2], z15, q[:, 2:3],
         jnp.zeros((bm, _PW - _GW - 33), F32)], axis=1)


def _cry_tables(prec_fea, wg, bg, wm, bmv, w2g, b2g, pw, bm):
    return pl.pallas_call(
        _cry_body,
        grid=(_N // bm,),
        in_specs=[pl.BlockSpec((bm, _FEA), lambda i: (i, 0)),
                  pl.BlockSpec((_FEA, _GW), lambda i: (0, 0)),
                  pl.BlockSpec((1, _GW), lambda i: (0, 0)),
                  pl.BlockSpec((_FEA, _GW), lambda i: (0, 0)),
                  pl.BlockSpec((1, _GW), lambda i: (0, 0)),
                  pl.BlockSpec((1, _GW), lambda i: (0, 0)),
                  pl.BlockSpec((1, _H), lambda i: (0, 0)),
                  pl.BlockSpec((bm, 1), lambda i: (i, 0))],
        out_specs=pl.BlockSpec((bm, _PW), lambda i: (i, 0)),
        out_shape=jax.ShapeDtypeStruct((_N, _PW), F32),
    )(prec_fea, wg, bg.reshape(1, _GW), wm, bmv.reshape(1, _GW),
      w2g.reshape(1, _GW), b2g.reshape(1, _H), pw)


def _out_body(x_ref, mask_ref, w1, b1, r1, w2, b2, r2, w3, b3, w4, b4, r4,
              wo, bo, o_ref):
    x = x_ref[...]
    h = jnp.maximum(jnp.dot(x, w1[...], preferred_element_type=F32) + b1[...], 0.0)
    x = h + jnp.dot(x, r1[...], preferred_element_type=F32)
    h = jnp.maximum(jnp.dot(x, w2[...], preferred_element_type=F32) + b2[...], 0.0)
    x = h + jnp.dot(x, r2[...], preferred_element_type=F32)
    h = jnp.maximum(jnp.dot(x, w3[...], preferred_element_type=F32) + b3[...], 0.0)
    x = h + x
    h = jnp.maximum(jnp.dot(x, w4[...], preferred_element_type=F32) + b4[...], 0.0)
    x = h + jnp.dot(x, r4[...], preferred_element_type=F32)
    out = jnp.dot(x, wo[...], preferred_element_type=F32) + bo[...]
    o_ref[...] = jnp.where(mask_ref[...] != 0.0, out,
                           jnp.full_like(out, -999.0))


def _out_mlp(react_fea, mask, outp):
    fcs = outp['fcs']
    res = outp['res']
    wo, bo = outp['fc_out']
    t = 81
    args = [react_fea, mask,
            fcs[0][0], fcs[0][1].reshape(1, 256), res[0],
            fcs[1][0], fcs[1][1].reshape(1, 512), res[1],
            fcs[2][0], fcs[2][1].reshape(1, 512),
            fcs[3][0], fcs[3][1].reshape(1, 256), res[3],
            wo, bo.reshape(1, t)]
    return pl.pallas_call(
        _out_body,
        out_shape=jax.ShapeDtypeStruct((_C, t), F32),
    )(*args)


def _scmesh():
    return plsc.VectorSubcoreMesh(core_axis_name="c", subcore_axis_name="s")


def _edge_sc_body(sa_hbm, nb_hbm, act_hbm, nbr2d_hbm, meta_hbm,
                  bnd_hbm, w2_hbm, out_hbm,
                  sa_v, arow_v, nbrs_v, acc_v, gtmp_v, mrow_v, idx_v,
                  bnd_v, w2_v, sem, sem2, sem3):
    w = lax.axis_index("c") * 16 + lax.axis_index("s")

    pltpu.sync_copy(bnd_hbm.at[w], bnd_v)
    bv = bnd_v[pl.ds(0, 16)]
    n_lo = bv[0]
    n_hi = bv[1]
    pltpu.sync_copy(w2_hbm, w2_v)

    def node_body(n, _):
        pltpu.sync_copy(meta_hbm.at[n], mrow_v)
        mv = mrow_v[pl.ds(0, 16)]
        r = mv[0]
        c0 = mv[1]
        c1 = mv[2]
        e0 = mv[3]
        e1 = mv[4]
        cp_s = pltpu.async_copy(sa_hbm.at[n], sa_v, sem2)
        cp_a = pltpu.async_copy(act_hbm.at[r], arow_v, sem3)
        z16 = jnp.zeros((16,), F32)
        for j in range(_PW // 16):
            acc_v[pl.ds(j * 16, 16)] = z16
        cp_s.wait()
        cp_a.wait()
        for j in range(_TW // 16):
            sa_v[pl.ds(j * 16, 16)] = (sa_v[pl.ds(j * 16, 16)]
                                       + arow_v[pl.ds(j * 16, 16)])

        def chunk_body(c, _c):
            pltpu.sync_copy(nbr2d_hbm.at[c], idx_v)
            ids = idx_v[pl.ds(0, 16)]
            pltpu.async_copy(nb_hbm.at[ids], nbrs_v, sem).wait()

            b2v_c = w2_v[_GW // 16, pl.ds(0, 16)]

            def lane_body(e, _l):
                eid = c * 16 + e

                @pl.when((eid >= e0) & (eid < e1))
                def _valid():
                    wv = nbrs_v[e, pl.ds(_TW, 16)]
                    for h in range(_H):
                        gacc = jnp.zeros((16,), F32)
                        for j in range(_HID // 16):
                            off = h * _HID + j * 16
                            x = sa_v[pl.ds(off, 16)] + nbrs_v[e, pl.ds(off, 16)]
                            x = jnp.maximum(x, 0.01 * x)
                            gacc = gacc + x * w2_v[h * 16 + j, pl.ds(0, 16)]
                        gtmp_v[pl.ds(0, 16)] = gacc
                        gv = gtmp_v[pl.ds(0, 16)]
                        s0 = (gv[0] + gv[1]) + (gv[2] + gv[3])
                        s1 = (gv[4] + gv[5]) + (gv[6] + gv[7])
                        s2 = (gv[8] + gv[9]) + (gv[10] + gv[11])
                        s3 = (gv[12] + gv[13]) + (gv[14] + gv[15])
                        gl = (s0 + s1) + (s2 + s3) + b2v_c[h]
                        qvec = wv * jnp.exp(jnp.full((16,), gl, F32))
                        doff = _GW + 16 * h
                        acc_v[pl.ds(doff, 16)] = acc_v[pl.ds(doff, 16)] + qvec
                        gtmp_v[pl.ds(0, 16)] = qvec
                        qv2 = gtmp_v[pl.ds(0, 16)]
                        q = qv2[0]
                        for j in range(_HID // 16):
                            off = _GW + h * _HID + j * 16
                            x = sa_v[pl.ds(off, 16)] + nbrs_v[e, pl.ds(off, 16)]
                            x = jnp.maximum(x, 0.01 * x)
                            aoff = h * _HID + j * 16
                            acc_v[pl.ds(aoff, 16)] = (acc_v[pl.ds(aoff, 16)]
                                                      + x * q)
                return _l

            lax.fori_loop(0, 16, lane_body, 0)
            return _c

        lax.fori_loop(c0, c1, chunk_body, 0)
        pltpu.sync_copy(acc_v, out_hbm.at[n])
        return _

    lax.fori_loop(n_lo, n_hi, node_body, 0)


def _edge_attention(sa_all, nb_all, act_all, nbr2d, meta, bnd, w2tab):
    k = functools.partial(
        pl.kernel, mesh=_scmesh(),
        out_type=jax.ShapeDtypeStruct((_N, _PW), F32),
        scratch_types=[pltpu.VMEM((_TW,), F32),
                       pltpu.VMEM((_TW,), F32),
                       pltpu.VMEM((16, _NBW), F32),
                       pltpu.VMEM((_PW,), F32),
                       pltpu.VMEM((16,), F32),
                       pltpu.VMEM((16,), I32),
                       pltpu.VMEM((16,), I32),
                       pltpu.VMEM((16,), I32),
                       pltpu.VMEM((_GW // 16 + 1, 16), F32),
                       pltpu.SemaphoreType.DMA,
                       pltpu.SemaphoreType.DMA,
                       pltpu.SemaphoreType.DMA],
    )(_edge_sc_body)
    return k(sa_all, nb_all, act_all, nbr2d, meta, bnd, w2tab)


def _segsum_sc_body(t_hbm, idx2d_hbm, meta_hbm, bnd_hbm, out_hbm,
                    rows_v, acc_v, mrow_v, idx_v, bnd_v, sem):
    w = lax.axis_index("c") * 16 + lax.axis_index("s")

    pltpu.sync_copy(bnd_hbm.at[w], bnd_v)
    bv = bnd_v[pl.ds(0, 16)]
    n_lo = bv[0]
    n_hi = bv[1]

    def seg_body(n, _):
        pltpu.sync_copy(meta_hbm.at[n], mrow_v)
        mv = mrow_v[pl.ds(0, 16)]
        c0 = mv[0]
        c1 = mv[1]
        e0 = mv[2]
        e1 = mv[3]
        z16 = jnp.zeros((16,), F32)
        for j in range(_PW // 16):
            acc_v[pl.ds(j * 16, 16)] = z16

        def chunk_body(c, _c):
            pltpu.sync_copy(idx2d_hbm.at[c], idx_v)
            ids = idx_v[pl.ds(0, 16)]
            pltpu.async_copy(t_hbm.at[ids], rows_v, sem).wait()

            def lane_body(e, _l):
                eid = c * 16 + e

                @pl.when((eid >= e0) & (eid < e1))
                def _valid():
                    for j in range(_PW // 16):
                        acc_v[pl.ds(j * 16, 16)] = (acc_v[pl.ds(j * 16, 16)]
                                                    + rows_v[e, pl.ds(j * 16, 16)])
                return _l

            lax.fori_loop(0, 16, lane_body, 0)
            return _c

        lax.fori_loop(c0, c1, chunk_body, 0)
        pltpu.sync_copy(acc_v, out_hbm.at[n])
        return _

    lax.fori_loop(n_lo, n_hi, seg_body, 0)


def _segsum(tbl, idx2d, meta, bnd):
    k = functools.partial(
        pl.kernel, mesh=_scmesh(),
        out_type=jax.ShapeDtypeStruct((_C, _PW), F32),
        scratch_types=[pltpu.VMEM((16, _PW), F32),
                       pltpu.VMEM((_PW,), F32),
                       pltpu.VMEM((16,), I32),
                       pltpu.VMEM((16,), I32),
                       pltpu.VMEM((16,), I32),
                       pltpu.SemaphoreType.DMA],
    )(_segsum_sc_body)
    return k(tbl, idx2d, meta, bnd)


def _seg_meta(idx_sorted, nseg, nitems):
    off = jnp.searchsorted(idx_sorted, jnp.arange(nseg + 1), side='left')
    off = off.astype(I32)
    e0 = off[:-1]
    e1 = off[1:]
    c0 = e0 // 16
    c1 = jnp.where(e1 > e0, (e1 + 15) // 16, c0)
    return off, c0, c1


def _worker_bnd(off, nseg):
    total = off[-1]
    tgt = (jnp.arange(_NWORK + 1) * total) // _NWORK
    bnd = jnp.searchsorted(off, tgt, side='left').astype(I32)
    bnd = bnd.at[0].set(0).at[_NWORK].set(nseg)
    rows = jnp.stack([bnd[:-1], bnd[1:]], axis=1)
    return jnp.pad(rows, ((0, 0), (0, 14)))


def _pack16(x, pad_val, rows):
    need = rows * 16 - x.shape[0]
    return jnp.pad(x, (0, need), constant_values=pad_val).reshape(rows, 16)


def kernel(prec_weights, orig_prec_fea, self_fea_idx, nbr_fea_idx,
           reaction_prec_idx, actions_padded, actions_len, prec_elem_mask,
           params):
    self_fea_idx = self_fea_idx.astype(I32)
    nbr_fea_idx = nbr_fea_idx.astype(I32)
    reaction_prec_idx = reaction_prec_idx.astype(I32)

    actions = _gru(actions_padded, actions_len.astype(I32), params['rnn'])

    emb = _mm(orig_prec_fea, params['W_emb'],
              jnp.zeros((_FEA - 1,), F32), 1000)
    prec_fea = jnp.concatenate([emb, prec_weights], axis=1)

    off, c0, c1 = _seg_meta(self_fea_idx, _N, _E)
    meta = jnp.stack([reaction_prec_idx, c0, c1, off[:-1], off[1:]], axis=1)
    meta = jnp.pad(meta, ((0, 0), (0, 11)))
    bnd = _worker_bnd(off, _N)
    nbr2d = _pack16(nbr_fea_idx, 0, _ECH)

    for heads in params['graphs']:
        ws_l, wn_l, wa_l, bs_l = [], [], [], []
        w2g_l, b2g_l, w2m_l, b2m_l = [], [], [], []
        for net in ('gate', 'msg'):
            for h in heads:
                (w1, b1), (w2, b2) = h[net]
                ws_l.append(w1[:_FEA])
                wn_l.append(w1[_FEA:2 * _FEA])
                wa_l.append(w1[2 * _FEA:])
                bs_l.append(b1)
                if net == 'gate':
                    w2g_l.append(w2[:, 0])
                    b2g_l.append(b2[0])
                else:
                    w2m_l.append(w2)
                    b2m_l.append(b2)
        ws = jnp.concatenate(ws_l, axis=1)
        wn = jnp.concatenate(wn_l, axis=1)
        wa = jnp.concatenate(wa_l, axis=1)
        bs = jnp.concatenate(bs_l)
        both = _mm(prec_fea, jnp.concatenate([ws, wn], axis=1),
                   jnp.concatenate([bs, jnp.zeros((_TW,), F32)]), 1000)
        sa_all = both[:, :_TW]
        nb_all = jnp.concatenate(
            [both[:, _TW:], prec_weights, jnp.zeros((_N, 127), F32)], axis=1)
        act_all = _mm(actions, wa, jnp.zeros((_TW,), F32), 1000)
        w2tab = jnp.concatenate(
            [jnp.concatenate(w2g_l).reshape(_GW // 16, 16),
             jnp.pad(jnp.stack(b2g_l), (0, 13)).reshape(1, 16)], axis=0)
        pooled = _edge_attention(sa_all, nb_all, act_all, nbr2d,
                                 meta, bnd, w2tab)
        w2cat = jnp.concatenate([w / _H for w in w2m_l], axis=0)
        b2cat = jnp.stack([b / _H for b in b2m_l], axis=0)
        prec_fea = _pool_update(pooled, w2cat, b2cat, prec_fea, 1000)

    wg_l, bg_l, wm_l, bm_l = [], [], [], []
    w2g_l, b2g_l, w2m_l, b2m_l = [], [], [], []
    for h in params['cry_pool']:
        (w1, b1), (w2, b2) = h['gate']
        wg_l.append(w1)
        bg_l.append(b1)
        w2g_l.append(w2[:, 0])
        b2g_l.append(b2[0])
        (w1, b1), (w2, b2) = h['msg']
        wm_l.append(w1)
        bm_l.append(b1)
        w2m_l.append(w2)
        b2m_l.append(b2)
    tbl = _cry_tables(prec_fea,
                      jnp.concatenate(wg_l, axis=1), jnp.concatenate(bg_l),
                      jnp.concatenate(wm_l, axis=1), jnp.concatenate(bm_l),
                      jnp.concatenate(w2g_l), jnp.stack(b2g_l),
                      prec_weights, 1000)

    roff, rc0, rc1 = _seg_meta(reaction_prec_idx, _C, _N)
    rmeta = jnp.pad(jnp.stack([rc0, rc1, roff[:-1], roff[1:]], axis=1),
                    ((0, 0), (0, 12)))
    rbnd = _worker_bnd(roff, _C)
    idx2d = jnp.arange(_N, dtype=I32).reshape(_NCH, 16)
    pooled2 = _segsum(tbl, idx2d, rmeta, rbnd)

    w2cat = jnp.concatenate([w / _H for w in w2m_l], axis=0)
    b2cat = jnp.stack([b / _H for b in b2m_l], axis=0)
    react_fea = _pool_update(pooled2, w2cat, b2cat,
                             jnp.zeros((_C, _FEA), F32), 1000)

    output = _out_mlp(react_fea, prec_elem_mask, params['out'])
    return output, react_fea


# speedup vs baseline: 1.7166x; 1.0203x over previous
import functools

import jax
import jax.numpy as jnp
from jax import lax
from jax.experimental import pallas as pl
from jax.experimental.pallas import tpu as pltpu
from jax.experimental.pallas import tpu_sc as plsc

F32 = jnp.float32
I32 = jnp.int32

_N = 10000
_E = 320000
_C = 2000
_FEA = 128
_AF = 32
_H = 3
_HID = 256
_GW = _H * _HID
_TW = 2 * _GW
_NBW = _TW + 128
_PW = _GW + 128
_ECH = _E // 16
_NCH = _N // 16
_NWORK = 32


def _mm_body(x_ref, w_ref, b_ref, o_ref):
    o_ref[...] = jnp.dot(x_ref[...], w_ref[...],
                         preferred_element_type=F32) + b_ref[...]


def _mm(x, w, b, bm):
    m, k = x.shape
    n = w.shape[1]
    grid = m // bm
    return pl.pallas_call(
        _mm_body,
        grid=(grid,),
        in_specs=[pl.BlockSpec((bm, k), lambda i: (i, 0)),
                  pl.BlockSpec((k, n), lambda i: (0, 0)),
                  pl.BlockSpec((1, n), lambda i: (0, 0))],
        out_specs=pl.BlockSpec((bm, n), lambda i: (i, 0)),
        out_shape=jax.ShapeDtypeStruct((m, n), F32),
    )(x, w, b.reshape(1, n))


def _gru_body(x_ref, len_ref, wih_ref, whh_ref, bih_ref, bhh_ref, o_ref):
    cdim = x_ref.shape[0]
    h = jnp.zeros((cdim, _AF), F32)
    out = jnp.zeros((cdim, _AF), F32)
    idx = jnp.clip(len_ref[...] - 1, 0, 9)
    wih = wih_ref[...]
    whh = whh_ref[...]
    bih = bih_ref[...]
    bhh = bhh_ref[...]
    for t in range(10):
        xt = x_ref[:, t, :]
        gi = jnp.dot(xt, wih, preferred_element_type=F32) + bih
        gh = jnp.dot(h, whh, preferred_element_type=F32) + bhh
        i_r, i_z, i_n = gi[:, :_AF], gi[:, _AF:2 * _AF], gi[:, 2 * _AF:]
        h_r, h_z, h_n = gh[:, :_AF], gh[:, _AF:2 * _AF], gh[:, 2 * _AF:]
        r = jax.nn.sigmoid(i_r + h_r)
        z = jax.nn.sigmoid(i_z + h_z)
        nn_ = jnp.tanh(i_n + r * h_n)
        h = (1.0 - z) * nn_ + z * h
        out = jnp.where(idx == t, h, out)
    o_ref[...] = out


def _gru(actions_padded, actions_len, rnn):
    w_ih, w_hh, b_ih, b_hh = rnn
    return pl.pallas_call(
        _gru_body,
        out_shape=jax.ShapeDtypeStruct((_C, _AF), F32),
    )(actions_padded, actions_len.reshape(_C, 1), w_ih, w_hh,
      b_ih.reshape(1, 3 * _AF), b_hh.reshape(1, 3 * _AF))


def _pool_body(p_ref, w2_ref, b2_ref, base_ref, o_ref):
    pooled = p_ref[:, :_GW]
    bm0 = pooled.shape[0]
    d = p_ref[:, _GW:_GW + _H * 16].reshape(bm0, _H, 16)[:, :, 0]
    scale = 1.0 / (d + 1e-13)
    ratio = d * scale
    bm = pooled.shape[0]
    ps = (pooled.reshape(bm, _H, _HID) * scale[:, :, None]).reshape(bm, _GW)
    o_ref[...] = (jnp.dot(ps, w2_ref[...], preferred_element_type=F32)
                  + jnp.dot(ratio, b2_ref[...], preferred_element_type=F32)
                  + base_ref[...])


def _pool_update(pooled, w2cat, b2cat, base, bm):
    m = pooled.shape[0]
    return pl.pallas_call(
        _pool_body,
        grid=(m // bm,),
        in_specs=[pl.BlockSpec((bm, _PW), lambda i: (i, 0)),
                  pl.BlockSpec((_GW, _FEA), lambda i: (0, 0)),
                  pl.BlockSpec((_H, _FEA), lambda i: (0, 0)),
                  pl.BlockSpec((bm, _FEA), lambda i: (i, 0))],
        out_specs=pl.BlockSpec((bm, _FEA), lambda i: (i, 0)),
        out_shape=jax.ShapeDtypeStruct((m, _FEA), F32),
    )(pooled, w2cat, b2cat, base)


def _cry_body(x_ref, wg_ref, bg_ref, wm_ref, bm_ref, w2g_ref, b2g_ref,
              pw_ref, o_ref):
    x = x_ref[...]
    bm = x.shape[0]
    g = jnp.dot(x, wg_ref[...], preferred_element_type=F32) + bg_ref[...]
    g = jnp.maximum(g, 0.01 * g)
    logit = jnp.sum(g.reshape(bm, _H, _HID) * w2g_ref[...].reshape(1, _H, _HID),
                    axis=2) + b2g_ref[...]
    q = pw_ref[...] * jnp.exp(logit)
    m_ = jnp.dot(x, wm_ref[...], preferred_element_type=F32) + bm_ref[...]
    m_ = jnp.maximum(m_, 0.01 * m_)
    m_ = (m_.reshape(bm, _H, _HID) * q[:, :, None]).reshape(bm, _GW)
    z15 = jnp.zeros((bm, 15), F32)
    o_ref[...] = jnp.concatenate(
        [m_, q[:, 0:1], z15, q[:, 1:2], z15, q[:, 2:3],
         jnp.zeros((bm, _PW - _GW - 33), F32)], axis=1)


def _cry_tables(prec_fea, wg, bg, wm, bmv, w2g, b2g, pw, bm):
    return pl.pallas_call(
        _cry_body,
        grid=(_N // bm,),
        in_specs=[pl.BlockSpec((bm, _FEA), lambda i: (i, 0)),
                  pl.BlockSpec((_FEA, _GW), lambda i: (0, 0)),
                  pl.BlockSpec((1, _GW), lambda i: (0, 0)),
                  pl.BlockSpec((_FEA, _GW), lambda i: (0, 0)),
                  pl.BlockSpec((1, _GW), lambda i: (0, 0)),
                  pl.BlockSpec((1, _GW), lambda i: (0, 0)),
                  pl.BlockSpec((1, _H), lambda i: (0, 0)),
                  pl.BlockSpec((bm, 1), lambda i: (i, 0))],
        out_specs=pl.BlockSpec((bm, _PW), lambda i: (i, 0)),
        out_shape=jax.ShapeDtypeStruct((_N, _PW), F32),
    )(prec_fea, wg, bg.reshape(1, _GW), wm, bmv.reshape(1, _GW),
      w2g.reshape(1, _GW), b2g.reshape(1, _H), pw)


def _out_body(x_ref, mask_ref, w1, b1, r1, w2, b2, r2, w3, b3, w4, b4, r4,
              wo, bo, o_ref):
    x = x_ref[...]
    h = jnp.maximum(jnp.dot(x, w1[...], preferred_element_type=F32) + b1[...], 0.0)
    x = h + jnp.dot(x, r1[...], preferred_element_type=F32)
    h = jnp.maximum(jnp.dot(x, w2[...], preferred_element_type=F32) + b2[...], 0.0)
    x = h + jnp.dot(x, r2[...], preferred_element_type=F32)
    h = jnp.maximum(jnp.dot(x, w3[...], preferred_element_type=F32) + b3[...], 0.0)
    x = h + x
    h = jnp.maximum(jnp.dot(x, w4[...], preferred_element_type=F32) + b4[...], 0.0)
    x = h + jnp.dot(x, r4[...], preferred_element_type=F32)
    out = jnp.dot(x, wo[...], preferred_element_type=F32) + bo[...]
    o_ref[...] = jnp.where(mask_ref[...] != 0.0, out,
                           jnp.full_like(out, -999.0))


def _out_mlp(react_fea, mask, outp):
    fcs = outp['fcs']
    res = outp['res']
    wo, bo = outp['fc_out']
    t = 81
    args = [react_fea, mask,
            fcs[0][0], fcs[0][1].reshape(1, 256), res[0],
            fcs[1][0], fcs[1][1].reshape(1, 512), res[1],
            fcs[2][0], fcs[2][1].reshape(1, 512),
            fcs[3][0], fcs[3][1].reshape(1, 256), res[3],
            wo, bo.reshape(1, t)]
    return pl.pallas_call(
        _out_body,
        out_shape=jax.ShapeDtypeStruct((_C, t), F32),
    )(*args)


def _scmesh():
    return plsc.VectorSubcoreMesh(core_axis_name="c", subcore_axis_name="s")


def _edge_sc_body(sa_hbm, nb_hbm, act_hbm, nbr2d_hbm, meta_hbm,
                  bnd_hbm, w2_hbm, out_hbm,
                  sa_v, arow_v, nbrs_v, nbrs2_v, acc_v, gtmp_v, mrow_v, idx_v,
                  bnd_v, w2_v, sem, sem2, sem3):
    w = lax.axis_index("c") * 16 + lax.axis_index("s")

    pltpu.sync_copy(bnd_hbm.at[w], bnd_v)
    bv = bnd_v[pl.ds(0, 16)]
    n_lo = bv[0]
    n_hi = bv[1]
    pltpu.sync_copy(w2_hbm, w2_v)

    def node_body(n, _):
        pltpu.sync_copy(meta_hbm.at[n], mrow_v)
        mv = mrow_v[pl.ds(0, 16)]
        r = mv[0]
        c0 = mv[1]
        c1 = mv[2]
        e0 = mv[3]
        e1 = mv[4]
        cp_s = pltpu.async_copy(sa_hbm.at[n], sa_v, sem2)
        cp_a = pltpu.async_copy(act_hbm.at[r], arow_v, sem3)
        z16 = jnp.zeros((16,), F32)
        for j in range(_PW // 16):
            acc_v[pl.ds(j * 16, 16)] = z16
        cp_s.wait()
        cp_a.wait()
        for j in range(_TW // 16):
            sa_v[pl.ds(j * 16, 16)] = (sa_v[pl.ds(j * 16, 16)]
                                       + arow_v[pl.ds(j * 16, 16)])

        b2v_c = w2_v[_GW // 16, pl.ds(0, 16)]

        def run_lanes(nbuf, c):
            def lane_body(e, _l):
                eid = c * 16 + e

                @pl.when((eid >= e0) & (eid < e1))
                def _valid():
                    wv = nbuf[e, pl.ds(_TW, 16)]
                    for h in range(_H):
                        gacc = jnp.zeros((16,), F32)
                        for j in range(_HID // 16):
                            off = h * _HID + j * 16
                            x = sa_v[pl.ds(off, 16)] + nbuf[e, pl.ds(off, 16)]
                            x = jnp.maximum(x, 0.01 * x)
                            gacc = gacc + x * w2_v[h * 16 + j, pl.ds(0, 16)]
                        gtmp_v[pl.ds(0, 16)] = gacc
                        gv = gtmp_v[pl.ds(0, 16)]
                        s0 = (gv[0] + gv[1]) + (gv[2] + gv[3])
                        s1 = (gv[4] + gv[5]) + (gv[6] + gv[7])
                        s2 = (gv[8] + gv[9]) + (gv[10] + gv[11])
                        s3 = (gv[12] + gv[13]) + (gv[14] + gv[15])
                        gl = (s0 + s1) + (s2 + s3) + b2v_c[h]
                        qvec = wv * jnp.exp(jnp.full((16,), gl, F32))
                        doff = _GW + 16 * h
                        acc_v[pl.ds(doff, 16)] = acc_v[pl.ds(doff, 16)] + qvec
                        gtmp_v[pl.ds(0, 16)] = qvec
                        qv2 = gtmp_v[pl.ds(0, 16)]
                        q = qv2[0]
                        for j in range(_HID // 16):
                            off = _GW + h * _HID + j * 16
                            x = sa_v[pl.ds(off, 16)] + nbuf[e, pl.ds(off, 16)]
                            x = jnp.maximum(x, 0.01 * x)
                            aoff = h * _HID + j * 16
                            acc_v[pl.ds(aoff, 16)] = (acc_v[pl.ds(aoff, 16)]
                                                      + x * q)
                return _l

            lax.fori_loop(0, 16, lane_body, 0)

        def pair_body(p, _c):
            ca = c0 + 2 * p
            cb = ca + 1
            pltpu.sync_copy(nbr2d_hbm.at[ca], idx_v)
            ids_a = idx_v[pl.ds(0, 16)]
            cp_a = pltpu.async_copy(nb_hbm.at[ids_a], nbrs_v, sem)
            pltpu.sync_copy(nbr2d_hbm.at[cb], idx_v)
            ids_b = idx_v[pl.ds(0, 16)]
            cp_b = pltpu.async_copy(nb_hbm.at[ids_b], nbrs2_v, sem2)
            cp_a.wait()
            run_lanes(nbrs_v, ca)
            cp_b.wait()

            @pl.when(cb < c1)
            def _do_b():
                run_lanes(nbrs2_v, cb)
            return _c

        lax.fori_loop(0, (c1 - c0 + 1) // 2, pair_body, 0)
        pltpu.sync_copy(acc_v, out_hbm.at[n])
        return _

    lax.fori_loop(n_lo, n_hi, node_body, 0)


def _edge_attention(sa_all, nb_all, act_all, nbr2d, meta, bnd, w2tab):
    k = functools.partial(
        pl.kernel, mesh=_scmesh(),
        out_type=jax.ShapeDtypeStruct((_N, _PW), F32),
        scratch_types=[pltpu.VMEM((_TW,), F32),
                       pltpu.VMEM((_TW,), F32),
                       pltpu.VMEM((16, _NBW), F32),
                       pltpu.VMEM((16, _NBW), F32),
                       pltpu.VMEM((_PW,), F32),
                       pltpu.VMEM((16,), F32),
                       pltpu.VMEM((16,), I32),
                       pltpu.VMEM((16,), I32),
                       pltpu.VMEM((16,), I32),
                       pltpu.VMEM((_GW // 16 + 1, 16), F32),
                       pltpu.SemaphoreType.DMA,
                       pltpu.SemaphoreType.DMA,
                       pltpu.SemaphoreType.DMA],
    )(_edge_sc_body)
    return k(sa_all, nb_all, act_all, nbr2d, meta, bnd, w2tab)


def _segsum_sc_body(t_hbm, idx2d_hbm, meta_hbm, bnd_hbm, out_hbm,
                    rows_v, acc_v, mrow_v, idx_v, bnd_v, sem):
    w = lax.axis_index("c") * 16 + lax.axis_index("s")

    pltpu.sync_copy(bnd_hbm.at[w], bnd_v)
    bv = bnd_v[pl.ds(0, 16)]
    n_lo = bv[0]
    n_hi = bv[1]

    def seg_body(n, _):
        pltpu.sync_copy(meta_hbm.at[n], mrow_v)
        mv = mrow_v[pl.ds(0, 16)]
        c0 = mv[0]
        c1 = mv[1]
        e0 = mv[2]
        e1 = mv[3]
        z16 = jnp.zeros((16,), F32)
        for j in range(_PW // 16):
            acc_v[pl.ds(j * 16, 16)] = z16

        def chunk_body(c, _c):
            pltpu.sync_copy(idx2d_hbm.at[c], idx_v)
            ids = idx_v[pl.ds(0, 16)]
            pltpu.async_copy(t_hbm.at[ids], rows_v, sem).wait()

            def lane_body(e, _l):
                eid = c * 16 + e

                @pl.when((eid >= e0) & (eid < e1))
                def _valid():
                    for j in range(_PW // 16):
                        acc_v[pl.ds(j * 16, 16)] = (acc_v[pl.ds(j * 16, 16)]
                                                    + rows_v[e, pl.ds(j * 16, 16)])
                return _l

            lax.fori_loop(0, 16, lane_body, 0)
            return _c

        lax.fori_loop(c0, c1, chunk_body, 0)
        pltpu.sync_copy(acc_v, out_hbm.at[n])
        return _

    lax.fori_loop(n_lo, n_hi, seg_body, 0)


def _segsum(tbl, idx2d, meta, bnd):
    k = functools.partial(
        pl.kernel, mesh=_scmesh(),
        out_type=jax.ShapeDtypeStruct((_C, _PW), F32),
        scratch_types=[pltpu.VMEM((16, _PW), F32),
                       pltpu.VMEM((_PW,), F32),
                       pltpu.VMEM((16,), I32),
                       pltpu.VMEM((16,), I32),
                       pltpu.VMEM((16,), I32),
                       pltpu.SemaphoreType.DMA],
    )(_segsum_sc_body)
    return k(tbl, idx2d, meta, bnd)


def _seg_meta(idx_sorted, nseg, nitems):
    off = jnp.searchsorted(idx_sorted, jnp.arange(nseg + 1), side='left')
    off = off.astype(I32)
    e0 = off[:-1]
    e1 = off[1:]
    c0 = e0 // 16
    c1 = jnp.where(e1 > e0, (e1 + 15) // 16, c0)
    return off, c0, c1


def _worker_bnd(off, nseg):
    total = off[-1]
    tgt = (jnp.arange(_NWORK + 1) * total) // _NWORK
    bnd = jnp.searchsorted(off, tgt, side='left').astype(I32)
    bnd = bnd.at[0].set(0).at[_NWORK].set(nseg)
    rows = jnp.stack([bnd[:-1], bnd[1:]], axis=1)
    return jnp.pad(rows, ((0, 0), (0, 14)))


def _pack16(x, pad_val, rows):
    need = rows * 16 - x.shape[0]
    return jnp.pad(x, (0, need), constant_values=pad_val).reshape(rows, 16)


def kernel(prec_weights, orig_prec_fea, self_fea_idx, nbr_fea_idx,
           reaction_prec_idx, actions_padded, actions_len, prec_elem_mask,
           params):
    self_fea_idx = self_fea_idx.astype(I32)
    nbr_fea_idx = nbr_fea_idx.astype(I32)
    reaction_prec_idx = reaction_prec_idx.astype(I32)

    actions = _gru(actions_padded, actions_len.astype(I32), params['rnn'])

    emb = _mm(orig_prec_fea, params['W_emb'],
              jnp.zeros((_FEA - 1,), F32), 1000)
    prec_fea = jnp.concatenate([emb, prec_weights], axis=1)

    off, c0, c1 = _seg_meta(self_fea_idx, _N, _E)
    meta = jnp.stack([reaction_prec_idx, c0, c1, off[:-1], off[1:]], axis=1)
    meta = jnp.pad(meta, ((0, 0), (0, 11)))
    bnd = _worker_bnd(off, _N)
    nbr2d = jnp.pad(_pack16(nbr_fea_idx, 0, _ECH), ((0, 1), (0, 0)))

    for heads in params['graphs']:
        ws_l, wn_l, wa_l, bs_l = [], [], [], []
        w2g_l, b2g_l, w2m_l, b2m_l = [], [], [], []
        for net in ('gate', 'msg'):
            for h in heads:
                (w1, b1), (w2, b2) = h[net]
                ws_l.append(w1[:_FEA])
                wn_l.append(w1[_FEA:2 * _FEA])
                wa_l.append(w1[2 * _FEA:])
                bs_l.append(b1)
                if net == 'gate':
                    w2g_l.append(w2[:, 0])
                    b2g_l.append(b2[0])
                else:
                    w2m_l.append(w2)
                    b2m_l.append(b2)
        ws = jnp.concatenate(ws_l, axis=1)
        wn = jnp.concatenate(wn_l, axis=1)
        wa = jnp.concatenate(wa_l, axis=1)
        bs = jnp.concatenate(bs_l)
        both = _mm(prec_fea, jnp.concatenate([ws, wn], axis=1),
                   jnp.concatenate([bs, jnp.zeros((_TW,), F32)]), 1000)
        sa_all = both[:, :_TW]
        nb_all = jnp.concatenate(
            [both[:, _TW:], prec_weights, jnp.zeros((_N, 127), F32)], axis=1)
        act_all = _mm(actions, wa, jnp.zeros((_TW,), F32), 1000)
        w2tab = jnp.concatenate(
            [jnp.concatenate(w2g_l).reshape(_GW // 16, 16),
             jnp.pad(jnp.stack(b2g_l), (0, 13)).reshape(1, 16)], axis=0)
        pooled = _edge_attention(sa_all, nb_all, act_all, nbr2d,
                                 meta, bnd, w2tab)
        w2cat = jnp.concatenate([w / _H for w in w2m_l], axis=0)
        b2cat = jnp.stack([b / _H for b in b2m_l], axis=0)
        prec_fea = _pool_update(pooled, w2cat, b2cat, prec_fea, 1000)

    wg_l, bg_l, wm_l, bm_l = [], [], [], []
    w2g_l, b2g_l, w2m_l, b2m_l = [], [], [], []
    for h in params['cry_pool']:
        (w1, b1), (w2, b2) = h['gate']
        wg_l.append(w1)
        bg_l.append(b1)
        w2g_l.append(w2[:, 0])
        b2g_l.append(b2[0])
        (w1, b1), (w2, b2) = h['msg']
        wm_l.append(w1)
        bm_l.append(b1)
        w2m_l.append(w2)
        b2m_l.append(b2)
    tbl = _cry_tables(prec_fea,
                      jnp.concatenate(wg_l, axis=1), jnp.concatenate(bg_l),
                      jnp.concatenate(wm_l, axis=1), jnp.concatenate(bm_l),
                      jnp.concatenate(w2g_l), jnp.stack(b2g_l),
                      prec_weights, 1000)

    roff, rc0, rc1 = _seg_meta(reaction_prec_idx, _C, _N)
    rmeta = jnp.pad(jnp.stack([rc0, rc1, roff[:-1], roff[1:]], axis=1),
                    ((0, 0), (0, 12)))
    rbnd = _worker_bnd(roff, _C)
    idx2d = jnp.arange(_N, dtype=I32).reshape(_NCH, 16)
    pooled2 = _segsum(tbl, idx2d, rmeta, rbnd)

    w2cat = jnp.concatenate([w / _H for w in w2m_l], axis=0)
    b2cat = jnp.stack([b / _H for b in b2m_l], axis=0)
    react_fea = _pool_update(pooled2, w2cat, b2cat,
                             jnp.zeros((_C, _FEA), F32), 1000)

    output = _out_mlp(react_fea, prec_elem_mask, params['out'])
    return output, react_fea
